# baseline (speedup 1.0000x reference)
import functools

import jax
import jax.numpy as jnp
from jax import lax
from jax.experimental import pallas as pl
from jax.experimental.pallas import tpu as pltpu
from jax.experimental.pallas import tpu_sc as plsc

N_ATOMS = 50000
N_EDGES = 1600000
N_FEATURES = 128
N_BASIS = 16
CUTOFF = 5.0

_NC, _NS = 2, 16
_NW = _NC * _NS
_B_PER_W = 1568
_B_PAD = _NW * _B_PER_W
_N_CHUNK = 2
_CHUNK = _B_PER_W // _N_CHUNK


@functools.partial(
    pl.kernel,
    out_type=jax.ShapeDtypeStruct((_B_PAD, N_FEATURES), jnp.float32),
    mesh=plsc.VectorSubcoreMesh(core_axis_name="c", subcore_axis_name="s"),
    scratch_types=[
        pltpu.VMEM((_B_PER_W,), jnp.int32),
        pltpu.VMEM((_CHUNK, N_FEATURES), jnp.float32),
        pltpu.SemaphoreType.DMA,
    ],
)
def _gather_kernel(table_hbm, idx_hbm, out_hbm, idx_v, rows_v, sem):
    wid = lax.axis_index("s") * _NC + lax.axis_index("c")
    base = wid * _B_PER_W
    pltpu.sync_copy(idx_hbm.at[pl.ds(base, _B_PER_W)], idx_v)
    for c in range(_N_CHUNK):
        pltpu.async_copy(table_hbm.at[idx_v.at[pl.ds(c * _CHUNK, _CHUNK)]],
                         rows_v, sem).wait()
        pltpu.sync_copy(rows_v, out_hbm.at[pl.ds(base + c * _CHUNK, _CHUNK)])


_BE = 2560
_DELTA = CUTOFF / (N_BASIS - 1)
_GAMMA = 1.0 / (2.0 * _DELTA * _DELTA)


def _edge_body(disp_ref, dir_ref, dist_ref):
    d = disp_ref[...]
    n2 = jnp.sum(d * d, axis=1, keepdims=True) + 1e-12
    dist = jnp.sqrt(n2)
    dir_ref[...] = d / dist
    cut = 0.5 * (jnp.cos((jnp.pi / CUTOFF) * dist) + 1.0)
    cut = jnp.where(dist < CUTOFF, cut, 0.0)
    centers = lax.broadcasted_iota(jnp.int32, (1, N_BASIS), 1).astype(jnp.float32) * _DELTA
    diff = dist - centers
    dist_ref[...] = cut * jnp.exp(-_GAMMA * (diff * diff))


_edge_call = pl.pallas_call(
    _edge_body,
    grid=(N_EDGES // _BE,),
    in_specs=[pl.BlockSpec((_BE, 3), lambda i: (i, 0))],
    out_specs=[
        pl.BlockSpec((_BE, 3), lambda i: (i, 0)),
        pl.BlockSpec((_BE, N_BASIS), lambda i: (i, 0)),
    ],
    out_shape=[
        jax.ShapeDtypeStruct((N_EDGES, 3), jnp.float32),
        jax.ShapeDtypeStruct((N_EDGES, N_BASIS), jnp.float32),
    ],
)


def kernel(z, disp, node_table):
    zi = jnp.pad(z.astype(jnp.int32), (0, _B_PAD - N_ATOMS))
    atom_node = _gather_kernel(node_table, zi)[:N_ATOMS]
    dir_edge, dist_edge = _edge_call(disp)
    force_node = jnp.zeros((N_ATOMS, 3, N_FEATURES), dtype=disp.dtype)
    disp_node = jnp.zeros((N_ATOMS, 3, N_FEATURES), dtype=disp.dtype)
    return (atom_node, force_node, disp_node, dir_edge, dist_edge)
